# Initial kernel scaffold; baseline (speedup 1.0000x reference)
import jax
import jax.numpy as jnp
from jax.experimental import pallas as pl


def kernel(x, router_w, gate_w, up_w, down_w):
    raise NotImplementedError("write your pallas kernel here")


# speedup vs baseline: 1.2623x; 1.2623x over previous
import functools

import jax
import jax.numpy as jnp
from jax import lax
from jax.experimental import pallas as pl
from jax.experimental.pallas import tpu as pltpu

EMBED = 1024
FF = 2048
NE = 8
TOPK = 2
T = 2048
NPAIR = T * TOPK
BT = 256
NB = NPAIR // BT + NE
NPAD = NB * BT
LANES = 128


def _router_body(x_ref, rw_ref, e1_ref, e2_ref, p1_ref, p2_ref):
    logits = jnp.dot(x_ref[...], rw_ref[...], preferred_element_type=jnp.float32)
    lane = lax.broadcasted_iota(jnp.int32, (T, LANES), 1)
    neg = jnp.float32(-1e30)
    logits = jnp.where(lane < NE, logits, neg)
    m1 = jnp.max(logits, axis=1, keepdims=True)
    i1 = jnp.min(jnp.where(logits == m1, lane, LANES), axis=1, keepdims=True)
    l2 = jnp.where(lane == i1, neg, logits)
    m2 = jnp.max(l2, axis=1, keepdims=True)
    i2 = jnp.min(jnp.where(l2 == m2, lane, LANES), axis=1, keepdims=True)
    p1 = 1.0 / (1.0 + jnp.exp(m2 - m1))
    p2 = 1.0 - p1
    zero = jnp.zeros((T, LANES), jnp.int32)
    e1_ref[...] = zero + i1
    e2_ref[...] = zero + i2
    p1_ref[...] = jnp.zeros((T, LANES), jnp.float32) + p1
    p2_ref[...] = jnp.zeros((T, LANES), jnp.float32) + p2


def _router(x_flat, router_w):
    rw_pad = jnp.zeros((EMBED, LANES), jnp.float32).at[:, :NE].set(router_w)
    outs = pl.pallas_call(
        _router_body,
        out_shape=(
            jax.ShapeDtypeStruct((T, LANES), jnp.int32),
            jax.ShapeDtypeStruct((T, LANES), jnp.int32),
            jax.ShapeDtypeStruct((T, LANES), jnp.float32),
            jax.ShapeDtypeStruct((T, LANES), jnp.float32),
        ),
    )(x_flat, rw_pad)
    e1, e2, p1, p2 = (o[:, 0] for o in outs)
    return e1, e2, p1, p2


def _plan(e1, e2):
    keys = jnp.concatenate([e1, e2])
    perm = jnp.argsort(keys, stable=True).astype(jnp.int32)
    sorted_e = keys[perm]
    counts = jnp.zeros((NE,), jnp.int32).at[keys].add(1)
    group_start = jnp.cumsum(counts) - counts
    nblk = (counts + BT - 1) // BT
    blk_start = jnp.cumsum(nblk) - nblk
    num_used = jnp.sum(nblk)
    padded_pos = (jnp.arange(NPAIR, dtype=jnp.int32)
                  + (blk_start[sorted_e] * BT - group_start[sorted_e]).astype(jnp.int32))
    row_token = jnp.zeros((NPAD,), jnp.int32).at[padded_pos].set(perm % T)
    pos_by_pair = jnp.zeros((NPAIR,), jnp.int32).at[perm].set(padded_pos)
    blk_end = jnp.cumsum(nblk)
    bidx = jnp.arange(NB, dtype=jnp.int32)
    block_expert = jnp.clip(
        jnp.searchsorted(blk_end, bidx, side="right"), 0, NE - 1
    ).astype(jnp.int32)
    block_active = (bidx < num_used).astype(jnp.int32)
    return row_token, padded_pos, perm, pos_by_pair, block_expert, block_active


def _sc_gather(x_flat, row_token):
    return jnp.take(x_flat, row_token, axis=0)


def _sc_combine(y, i0, i1):
    return jnp.take(y, i0, axis=0) + jnp.take(y, i1, axis=0)


def _gemm_body(be_ref, ba_ref, x_ref, gw_ref, uw_ref, dw_ref, p_ref, y_ref):
    b = pl.program_id(0)

    @pl.when(ba_ref[b] == 1)
    def _():
        x = x_ref[...]
        g = jnp.dot(x, gw_ref[0], preferred_element_type=jnp.float32)
        u = jnp.dot(x, uw_ref[0], preferred_element_type=jnp.float32)
        h = (g * lax.logistic(g)) * u
        y = jnp.dot(h, dw_ref[0], preferred_element_type=jnp.float32)
        y_ref[...] = y * p_ref[:, 0:1]


def _grouped_gemm(x_s, gate_w, up_w, down_w, row_p2d, block_expert, block_active):
    grid_spec = pltpu.PrefetchScalarGridSpec(
        num_scalar_prefetch=2,
        grid=(NB,),
        in_specs=[
            pl.BlockSpec((BT, EMBED), lambda b, be, ba: (b, 0)),
            pl.BlockSpec((1, EMBED, FF), lambda b, be, ba: (be[b], 0, 0)),
            pl.BlockSpec((1, EMBED, FF), lambda b, be, ba: (be[b], 0, 0)),
            pl.BlockSpec((1, FF, EMBED), lambda b, be, ba: (be[b], 0, 0)),
            pl.BlockSpec((BT, LANES), lambda b, be, ba: (b, 0)),
        ],
        out_specs=pl.BlockSpec((BT, EMBED), lambda b, be, ba: (b, 0)),
    )
    return pl.pallas_call(
        _gemm_body,
        grid_spec=grid_spec,
        out_shape=jax.ShapeDtypeStruct((NPAD, EMBED), jnp.float32),
        compiler_params=pltpu.CompilerParams(
            dimension_semantics=("arbitrary",),
        ),
    )(block_expert, block_active, x_s, gate_w, up_w, down_w, row_p2d)


def kernel(x, router_w, gate_w, up_w, down_w):
    B, S, D = x.shape
    x_flat = x.reshape(T, D)
    e1, e2, p1, p2 = _router(x_flat, router_w)
    row_token, padded_pos, perm, pos_by_pair, block_expert, block_active = _plan(e1, e2)
    p_all = jnp.concatenate([p1, p2])
    row_p = jnp.zeros((NPAD,), jnp.float32).at[padded_pos].set(p_all[perm])
    row_p2d = jnp.broadcast_to(row_p[:, None], (NPAD, LANES))
    x_s = _sc_gather(x_flat, row_token)
    y = _grouped_gemm(x_s, gate_w, up_w, down_w, row_p2d, block_expert, block_active)
    out = _sc_combine(y, pos_by_pair[:T], pos_by_pair[T:])
    return out.reshape(B, S, D)


# speedup vs baseline: 1.8905x; 1.4976x over previous
import functools

import jax
import jax.numpy as jnp
from jax import lax
from jax.experimental import pallas as pl
from jax.experimental.pallas import tpu as pltpu
from jax.experimental.pallas import tpu_sc as plsc

EMBED = 1024
FF = 2048
NE = 8
TOPK = 2
T = 2048
NPAIR = T * TOPK
BT = 256
NB = NPAIR // BT + NE
NPAD = NB * BT
LANES = 128


def _router_body(x_ref, rw_ref, e1_ref, e2_ref, p1_ref, p2_ref):
    logits = jnp.dot(x_ref[...], rw_ref[...], preferred_element_type=jnp.float32)
    lane = lax.broadcasted_iota(jnp.int32, (T, LANES), 1)
    neg = jnp.float32(-1e30)
    logits = jnp.where(lane < NE, logits, neg)
    m1 = jnp.max(logits, axis=1, keepdims=True)
    i1 = jnp.min(jnp.where(logits == m1, lane, LANES), axis=1, keepdims=True)
    l2 = jnp.where(lane == i1, neg, logits)
    m2 = jnp.max(l2, axis=1, keepdims=True)
    i2 = jnp.min(jnp.where(l2 == m2, lane, LANES), axis=1, keepdims=True)
    p1 = 1.0 / (1.0 + jnp.exp(m2 - m1))
    p2 = 1.0 - p1
    zero = jnp.zeros((T, LANES), jnp.int32)
    e1_ref[...] = zero + i1
    e2_ref[...] = zero + i2
    p1_ref[...] = jnp.zeros((T, LANES), jnp.float32) + p1
    p2_ref[...] = jnp.zeros((T, LANES), jnp.float32) + p2


def _router(x_flat, router_w):
    rw_pad = jnp.zeros((EMBED, LANES), jnp.float32).at[:, :NE].set(router_w)
    outs = pl.pallas_call(
        _router_body,
        out_shape=(
            jax.ShapeDtypeStruct((T, LANES), jnp.int32),
            jax.ShapeDtypeStruct((T, LANES), jnp.int32),
            jax.ShapeDtypeStruct((T, LANES), jnp.float32),
            jax.ShapeDtypeStruct((T, LANES), jnp.float32),
        ),
    )(x_flat, rw_pad)
    e1, e2, p1, p2 = (o[:, 0] for o in outs)
    return e1, e2, p1, p2


def _plan(e1, e2):
    keys = jnp.concatenate([e1, e2])
    onehot = (keys[:, None] == jnp.arange(NE, dtype=jnp.int32)[None, :]).astype(jnp.int32)
    csum = jnp.cumsum(onehot, axis=0)
    rank = jnp.sum(onehot * csum, axis=1) - 1
    counts = csum[-1]
    nblk = (counts + BT - 1) // BT
    blk_start = jnp.cumsum(nblk) - nblk
    num_used = jnp.sum(nblk)
    pos_by_pair = (jnp.sum(onehot * blk_start[None, :], axis=1) * BT + rank).astype(jnp.int32)
    blk_end = jnp.cumsum(nblk)
    bidx = jnp.arange(NB, dtype=jnp.int32)
    block_expert = jnp.clip(
        jnp.sum((blk_end[None, :] <= bidx[:, None]).astype(jnp.int32), axis=1),
        0, NE - 1).astype(jnp.int32)
    block_active = (bidx < num_used).astype(jnp.int32)
    return pos_by_pair, block_expert, block_active


NW = 32
DCH = 64
CCH = 32


@functools.lru_cache(maxsize=None)
def _sc_dispatch_k():
    mesh = plsc.VectorSubcoreMesh(core_axis_name="c", subcore_axis_name="s")

    @functools.partial(
        pl.kernel,
        mesh=mesh,
        out_type=jax.ShapeDtypeStruct((NPAD, EMBED), jnp.float32),
        scratch_types=[
            pltpu.VMEM((DCH,), jnp.int32),
            pltpu.VMEM((DCH,), jnp.int32),
            pltpu.VMEM((DCH, EMBED), jnp.float32),
            pltpu.SemaphoreType.DMA,
            pltpu.SemaphoreType.DMA,
        ],
    )
    def k(x_hbm, tok_hbm, pos_hbm, xs_hbm, tok_v, pos_v, rows_v, g_sem, s_sem):
        wid = lax.axis_index("s") * 2 + lax.axis_index("c")
        per_w = NPAIR // NW
        for c in range(per_w // DCH):
            base = wid * per_w + c * DCH
            pltpu.sync_copy(tok_hbm.at[pl.ds(base, DCH)], tok_v)
            pltpu.sync_copy(pos_hbm.at[pl.ds(base, DCH)], pos_v)
            pltpu.async_copy(x_hbm.at[tok_v], rows_v, g_sem).wait()
            pltpu.async_copy(rows_v, xs_hbm.at[pos_v], s_sem).wait()

    return k


@functools.lru_cache(maxsize=None)
def _sc_combine_k():
    mesh = plsc.VectorSubcoreMesh(core_axis_name="c", subcore_axis_name="s")

    @functools.partial(
        pl.kernel,
        mesh=mesh,
        out_type=jax.ShapeDtypeStruct((T, EMBED), jnp.float32),
        scratch_types=[
            pltpu.VMEM((CCH,), jnp.int32),
            pltpu.VMEM((CCH, EMBED), jnp.float32),
            pltpu.VMEM((CCH, EMBED), jnp.float32),
            pltpu.SemaphoreType.DMA,
            pltpu.SemaphoreType.DMA,
        ],
    )
    def k(y_hbm, i0_hbm, i1_hbm, out_hbm, idx_v, a_v, b_v, sem_a, sem_b):
        wid = lax.axis_index("s") * 2 + lax.axis_index("c")
        per_w = T // NW
        for c in range(per_w // CCH):
            base = wid * per_w + c * CCH
            pltpu.sync_copy(i0_hbm.at[pl.ds(base, CCH)], idx_v)
            pltpu.async_copy(y_hbm.at[idx_v], a_v, sem_a).wait()
            pltpu.sync_copy(i1_hbm.at[pl.ds(base, CCH)], idx_v)
            pltpu.async_copy(y_hbm.at[idx_v], b_v, sem_b).wait()

            def add_body(i, _):
                r = i // (EMBED // 16)
                col = (i % (EMBED // 16)) * 16
                a_v[r, pl.ds(col, 16)] = a_v[r, pl.ds(col, 16)] + b_v[r, pl.ds(col, 16)]
                return 0

            lax.fori_loop(0, CCH * (EMBED // 16), add_body, 0)
            pltpu.sync_copy(a_v, out_hbm.at[pl.ds(base, CCH)])

    return k


def _sc_gather(x_flat, tok_pair, pos_by_pair):
    return _sc_dispatch_k()(x_flat, tok_pair, pos_by_pair)


def _sc_combine(y, i0, i1):
    return _sc_combine_k()(y, i0, i1)


def _gemm_body(be_ref, ba_ref, x_ref, gw_ref, uw_ref, dw_ref, p_ref, y_ref):
    b = pl.program_id(0)

    @pl.when(ba_ref[b] == 1)
    def _():
        x = x_ref[...]
        g = jnp.dot(x, gw_ref[0], preferred_element_type=jnp.float32)
        u = jnp.dot(x, uw_ref[0], preferred_element_type=jnp.float32)
        h = (g * lax.logistic(g)) * u
        y = jnp.dot(h, dw_ref[0], preferred_element_type=jnp.float32)
        y_ref[...] = y * p_ref[:, 0:1]


def _grouped_gemm(x_s, gate_w, up_w, down_w, row_p2d, block_expert, block_active):
    grid_spec = pltpu.PrefetchScalarGridSpec(
        num_scalar_prefetch=2,
        grid=(NB,),
        in_specs=[
            pl.BlockSpec((BT, EMBED), lambda b, be, ba: (b, 0)),
            pl.BlockSpec((1, EMBED, FF), lambda b, be, ba: (be[b], 0, 0)),
            pl.BlockSpec((1, EMBED, FF), lambda b, be, ba: (be[b], 0, 0)),
            pl.BlockSpec((1, FF, EMBED), lambda b, be, ba: (be[b], 0, 0)),
            pl.BlockSpec((BT, LANES), lambda b, be, ba: (b, 0)),
        ],
        out_specs=pl.BlockSpec((BT, EMBED), lambda b, be, ba: (b, 0)),
    )
    return pl.pallas_call(
        _gemm_body,
        grid_spec=grid_spec,
        out_shape=jax.ShapeDtypeStruct((NPAD, EMBED), jnp.float32),
        compiler_params=pltpu.CompilerParams(
            dimension_semantics=("arbitrary",),
        ),
    )(block_expert, block_active, x_s, gate_w, up_w, down_w, row_p2d)


def kernel(x, router_w, gate_w, up_w, down_w):
    B, S, D = x.shape
    x_flat = x.reshape(T, D)
    e1, e2, p1, p2 = _router(x_flat, router_w)
    pos_by_pair, block_expert, block_active = _plan(e1, e2)
    p_all = jnp.concatenate([p1, p2])
    row_p = jnp.zeros((NPAD,), jnp.float32).at[pos_by_pair].set(p_all)
    row_p2d = jnp.broadcast_to(row_p[:, None], (NPAD, LANES))
    tok_pair = jnp.tile(jnp.arange(T, dtype=jnp.int32), 2)
    x_s = _sc_gather(x_flat, tok_pair, pos_by_pair)
    y = _grouped_gemm(x_s, gate_w, up_w, down_w, row_p2d, block_expert, block_active)
    out = _sc_combine(y, pos_by_pair[:T], pos_by_pair[T:])
    return out.reshape(B, S, D)


# speedup vs baseline: 2.0086x; 1.0625x over previous
import functools

import jax
import jax.numpy as jnp
from jax import lax
from jax.experimental import pallas as pl
from jax.experimental.pallas import tpu as pltpu
from jax.experimental.pallas import tpu_sc as plsc

EMBED = 1024
FF = 2048
NE = 8
TOPK = 2
T = 2048
NPAIR = T * TOPK
BT = 256
NB = NPAIR // BT + NE
NPAD = NB * BT
LANES = 128


def _router_body(x_ref, rw_ref, e1_ref, e2_ref, p1_ref, p2_ref):
    logits = jnp.dot(x_ref[...], rw_ref[...], preferred_element_type=jnp.float32)
    lane = lax.broadcasted_iota(jnp.int32, (T, LANES), 1)
    neg = jnp.float32(-1e30)
    logits = jnp.where(lane < NE, logits, neg)
    m1 = jnp.max(logits, axis=1, keepdims=True)
    i1 = jnp.min(jnp.where(logits == m1, lane, LANES), axis=1, keepdims=True)
    l2 = jnp.where(lane == i1, neg, logits)
    m2 = jnp.max(l2, axis=1, keepdims=True)
    i2 = jnp.min(jnp.where(l2 == m2, lane, LANES), axis=1, keepdims=True)
    p1 = 1.0 / (1.0 + jnp.exp(m2 - m1))
    p2 = 1.0 - p1
    zero = jnp.zeros((T, LANES), jnp.int32)
    e1_ref[...] = zero + i1
    e2_ref[...] = zero + i2
    p1_ref[...] = jnp.zeros((T, LANES), jnp.float32) + p1
    p2_ref[...] = jnp.zeros((T, LANES), jnp.float32) + p2


def _router(x_flat, router_w):
    rw_pad = jnp.zeros((EMBED, LANES), jnp.float32).at[:, :NE].set(router_w)
    outs = pl.pallas_call(
        _router_body,
        out_shape=(
            jax.ShapeDtypeStruct((T, LANES), jnp.int32),
            jax.ShapeDtypeStruct((T, LANES), jnp.int32),
            jax.ShapeDtypeStruct((T, LANES), jnp.float32),
            jax.ShapeDtypeStruct((T, LANES), jnp.float32),
        ),
    )(x_flat, rw_pad)
    e1, e2, p1, p2 = (o[:, 0] for o in outs)
    return e1, e2, p1, p2


def _plan(e1, e2):
    keys = jnp.concatenate([e1, e2])
    onehot = (keys[:, None] == jnp.arange(NE, dtype=jnp.int32)[None, :]).astype(jnp.int32)
    csum = jnp.cumsum(onehot, axis=0)
    rank = jnp.sum(onehot * csum, axis=1) - 1
    counts = csum[-1]
    nblk = (counts + BT - 1) // BT
    blk_start = jnp.cumsum(nblk) - nblk
    num_used = jnp.sum(nblk)
    pos_by_pair = (jnp.sum(onehot * blk_start[None, :], axis=1) * BT + rank).astype(jnp.int32)
    blk_end = jnp.cumsum(nblk)
    bidx = jnp.arange(NB, dtype=jnp.int32)
    block_expert = jnp.clip(
        jnp.sum((blk_end[None, :] <= bidx[:, None]).astype(jnp.int32), axis=1),
        0, NE - 1).astype(jnp.int32)
    block_active = (bidx < num_used).astype(jnp.int32)
    return pos_by_pair, block_expert, block_active


NW = 32
DCH = 64
CCH = 32


@functools.lru_cache(maxsize=None)
def _sc_dispatch_k():
    mesh = plsc.VectorSubcoreMesh(core_axis_name="c", subcore_axis_name="s")

    @functools.partial(
        pl.kernel,
        mesh=mesh,
        out_type=jax.ShapeDtypeStruct((NPAD, EMBED), jnp.float32),
        scratch_types=[
            pltpu.VMEM((DCH,), jnp.int32),
            pltpu.VMEM((DCH,), jnp.int32),
            pltpu.VMEM((DCH, EMBED), jnp.float32),
            pltpu.SemaphoreType.DMA,
            pltpu.SemaphoreType.DMA,
        ],
    )
    def k(x_hbm, tok_hbm, pos_hbm, xs_hbm, tok_v, pos_v, rows_v, g_sem, s_sem):
        wid = lax.axis_index("s") * 2 + lax.axis_index("c")
        per_w = NPAIR // NW
        for c in range(per_w // DCH):
            base = wid * per_w + c * DCH
            pltpu.sync_copy(tok_hbm.at[pl.ds(base, DCH)], tok_v)
            pltpu.sync_copy(pos_hbm.at[pl.ds(base, DCH)], pos_v)
            pltpu.async_copy(x_hbm.at[tok_v], rows_v, g_sem).wait()
            pltpu.async_copy(rows_v, xs_hbm.at[pos_v], s_sem).wait()

    return k


@functools.lru_cache(maxsize=None)
def _sc_combine_k():
    mesh = plsc.VectorSubcoreMesh(core_axis_name="c", subcore_axis_name="s")

    @functools.partial(
        pl.kernel,
        mesh=mesh,
        out_type=(
            jax.ShapeDtypeStruct((T, EMBED), jnp.float32),
            jax.ShapeDtypeStruct((T, EMBED), jnp.float32),
        ),
        scratch_types=[
            pltpu.VMEM((CCH,), jnp.int32),
            pltpu.VMEM((CCH, EMBED), jnp.float32),
            pltpu.VMEM((CCH, EMBED), jnp.float32),
            pltpu.SemaphoreType.DMA,
            pltpu.SemaphoreType.DMA,
        ],
    )
    def k(y_hbm, i0_hbm, i1_hbm, o0_hbm, o1_hbm, idx_v, a_v, b_v, sem_a, sem_b):
        wid = lax.axis_index("s") * 2 + lax.axis_index("c")
        per_w = T // NW
        for c in range(per_w // CCH):
            base = wid * per_w + c * CCH
            pltpu.sync_copy(i0_hbm.at[pl.ds(base, CCH)], idx_v)
            pltpu.async_copy(y_hbm.at[idx_v], a_v, sem_a).wait()
            pltpu.sync_copy(i1_hbm.at[pl.ds(base, CCH)], idx_v)
            pltpu.async_copy(y_hbm.at[idx_v], b_v, sem_b).wait()
            pltpu.sync_copy(a_v, o0_hbm.at[pl.ds(base, CCH)])
            pltpu.sync_copy(b_v, o1_hbm.at[pl.ds(base, CCH)])

    return k


def _sc_gather(x_flat, tok_pair, pos_by_pair):
    return _sc_dispatch_k()(x_flat, tok_pair, pos_by_pair)


def _sc_combine(y, i0, i1):
    return _sc_combine_k()(y, i0, i1)


def _gemm_body(be_ref, ba_ref, x_ref, gw_ref, uw_ref, dw_ref, y_ref):
    b = pl.program_id(0)

    @pl.when(ba_ref[b] == 1)
    def _():
        x = x_ref[...]
        g = jnp.dot(x, gw_ref[0], preferred_element_type=jnp.float32)
        u = jnp.dot(x, uw_ref[0], preferred_element_type=jnp.float32)
        h = (g * lax.logistic(g)) * u
        y_ref[...] = jnp.dot(h, dw_ref[0], preferred_element_type=jnp.float32)


def _grouped_gemm(x_s, gate_w, up_w, down_w, block_expert, block_active):
    grid_spec = pltpu.PrefetchScalarGridSpec(
        num_scalar_prefetch=2,
        grid=(NB,),
        in_specs=[
            pl.BlockSpec((BT, EMBED), lambda b, be, ba: (b, 0)),
            pl.BlockSpec((1, EMBED, FF), lambda b, be, ba: (be[b], 0, 0)),
            pl.BlockSpec((1, EMBED, FF), lambda b, be, ba: (be[b], 0, 0)),
            pl.BlockSpec((1, FF, EMBED), lambda b, be, ba: (be[b], 0, 0)),
        ],
        out_specs=pl.BlockSpec((BT, EMBED), lambda b, be, ba: (b, 0)),
    )
    return pl.pallas_call(
        _gemm_body,
        grid_spec=grid_spec,
        out_shape=jax.ShapeDtypeStruct((NPAD, EMBED), jnp.float32),
        compiler_params=pltpu.CompilerParams(
            dimension_semantics=("arbitrary",),
        ),
    )(block_expert, block_active, x_s, gate_w, up_w, down_w)


def kernel(x, router_w, gate_w, up_w, down_w):
    B, S, D = x.shape
    x_flat = x.reshape(T, D)
    e1, e2, p1, p2 = _router(x_flat, router_w)
    pos_by_pair, block_expert, block_active = _plan(e1, e2)
    tok_pair = jnp.tile(jnp.arange(T, dtype=jnp.int32), 2)
    x_s = _sc_gather(x_flat, tok_pair, pos_by_pair)
    y = _grouped_gemm(x_s, gate_w, up_w, down_w, block_expert, block_active)
    g0, g1 = _sc_combine(y, pos_by_pair[:T], pos_by_pair[T:])
    out = p1[:, None] * g0 + p2[:, None] * g1
    return out.reshape(B, S, D)


# speedup vs baseline: 2.0338x; 1.0125x over previous
import functools

import jax
import jax.numpy as jnp
from jax import lax
from jax.experimental import pallas as pl
from jax.experimental.pallas import tpu as pltpu
from jax.experimental.pallas import tpu_sc as plsc

EMBED = 1024
FF = 2048
NE = 8
TOPK = 2
T = 2048
NPAIR = T * TOPK
BT = 256
NB = NPAIR // BT + NE
NPAD = NB * BT
LANES = 128


def _router_body(x_ref, rw_ref, e1_ref, e2_ref, p1_ref, p2_ref):
    logits = jnp.dot(x_ref[...], rw_ref[...], preferred_element_type=jnp.float32)
    lane = lax.broadcasted_iota(jnp.int32, (T, LANES), 1)
    neg = jnp.float32(-1e30)
    logits = jnp.where(lane < NE, logits, neg)
    m1 = jnp.max(logits, axis=1, keepdims=True)
    i1 = jnp.min(jnp.where(logits == m1, lane, LANES), axis=1, keepdims=True)
    l2 = jnp.where(lane == i1, neg, logits)
    m2 = jnp.max(l2, axis=1, keepdims=True)
    i2 = jnp.min(jnp.where(l2 == m2, lane, LANES), axis=1, keepdims=True)
    p1 = 1.0 / (1.0 + jnp.exp(m2 - m1))
    p2 = 1.0 - p1
    zero = jnp.zeros((T, LANES), jnp.int32)
    e1_ref[...] = zero + i1
    e2_ref[...] = zero + i2
    p1_ref[...] = jnp.zeros((T, LANES), jnp.float32) + p1
    p2_ref[...] = jnp.zeros((T, LANES), jnp.float32) + p2


def _router(x_flat, router_w):
    rw_pad = jnp.zeros((EMBED, LANES), jnp.float32).at[:, :NE].set(router_w)
    outs = pl.pallas_call(
        _router_body,
        out_shape=(
            jax.ShapeDtypeStruct((T, LANES), jnp.int32),
            jax.ShapeDtypeStruct((T, LANES), jnp.int32),
            jax.ShapeDtypeStruct((T, LANES), jnp.float32),
            jax.ShapeDtypeStruct((T, LANES), jnp.float32),
        ),
    )(x_flat, rw_pad)
    e1, e2, p1, p2 = (o[:, 0] for o in outs)
    return e1, e2, p1, p2


def _plan(e1, e2):
    keys = jnp.concatenate([e1, e2])
    onehot = (keys[:, None] == jnp.arange(NE, dtype=jnp.int32)[None, :]).astype(jnp.int32)
    csum = jnp.cumsum(onehot, axis=0)
    rank = jnp.sum(onehot * csum, axis=1) - 1
    counts = csum[-1]
    nblk = (counts + BT - 1) // BT
    blk_start = jnp.cumsum(nblk) - nblk
    num_used = jnp.sum(nblk)
    pos_by_pair = (jnp.sum(onehot * blk_start[None, :], axis=1) * BT + rank).astype(jnp.int32)
    blk_end = jnp.cumsum(nblk)
    bidx = jnp.arange(NB, dtype=jnp.int32)
    block_expert = jnp.clip(
        jnp.sum((blk_end[None, :] <= bidx[:, None]).astype(jnp.int32), axis=1),
        0, NE - 1).astype(jnp.int32)
    block_active = (bidx < num_used).astype(jnp.int32)
    return pos_by_pair, block_expert, block_active


NW = 32
DCH = 64
CCH = 32


@functools.lru_cache(maxsize=None)
def _sc_dispatch_k():
    mesh = plsc.VectorSubcoreMesh(core_axis_name="c", subcore_axis_name="s")

    @functools.partial(
        pl.kernel,
        mesh=mesh,
        out_type=jax.ShapeDtypeStruct((NPAD, EMBED), jnp.float32),
        scratch_types=[
            pltpu.VMEM((DCH,), jnp.int32),
            pltpu.VMEM((DCH,), jnp.int32),
            pltpu.VMEM((DCH, EMBED), jnp.float32),
            pltpu.SemaphoreType.DMA,
            pltpu.SemaphoreType.DMA,
        ],
    )
    def k(x_hbm, tok_hbm, pos_hbm, xs_hbm, tok_v, pos_v, rows_v, g_sem, s_sem):
        wid = lax.axis_index("s") * 2 + lax.axis_index("c")
        per_w = NPAIR // NW
        for c in range(per_w // DCH):
            base = wid * per_w + c * DCH
            pltpu.sync_copy(tok_hbm.at[pl.ds(base, DCH)], tok_v)
            pltpu.sync_copy(pos_hbm.at[pl.ds(base, DCH)], pos_v)
            pltpu.async_copy(x_hbm.at[tok_v], rows_v, g_sem).wait()
            pltpu.async_copy(rows_v, xs_hbm.at[pos_v], s_sem).wait()

    return k


@functools.lru_cache(maxsize=None)
def _sc_combine_k():
    mesh = plsc.VectorSubcoreMesh(core_axis_name="c", subcore_axis_name="s")

    @functools.partial(
        pl.kernel,
        mesh=mesh,
        out_type=(
            jax.ShapeDtypeStruct((T, EMBED), jnp.float32),
            jax.ShapeDtypeStruct((T, EMBED), jnp.float32),
        ),
        scratch_types=[
            pltpu.VMEM((CCH,), jnp.int32),
            pltpu.VMEM((CCH, EMBED), jnp.float32),
            pltpu.VMEM((CCH, EMBED), jnp.float32),
            pltpu.SemaphoreType.DMA,
            pltpu.SemaphoreType.DMA,
        ],
    )
    def k(y_hbm, i0_hbm, i1_hbm, o0_hbm, o1_hbm, idx_v, a_v, b_v, sem_a, sem_b):
        wid = lax.axis_index("s") * 2 + lax.axis_index("c")
        per_w = T // NW
        for c in range(per_w // CCH):
            base = wid * per_w + c * CCH
            pltpu.sync_copy(i0_hbm.at[pl.ds(base, CCH)], idx_v)
            pltpu.async_copy(y_hbm.at[idx_v], a_v, sem_a).wait()
            pltpu.sync_copy(i1_hbm.at[pl.ds(base, CCH)], idx_v)
            pltpu.async_copy(y_hbm.at[idx_v], b_v, sem_b).wait()
            pltpu.sync_copy(a_v, o0_hbm.at[pl.ds(base, CCH)])
            pltpu.sync_copy(b_v, o1_hbm.at[pl.ds(base, CCH)])

    return k


def _sc_gather(x_flat, tok_pair, pos_by_pair):
    return _sc_dispatch_k()(x_flat, tok_pair, pos_by_pair)


def _sc_combine(y, i0, i1):
    return _sc_combine_k()(y, i0, i1)


def _gemm_body(be_ref, ba_ref, x_ref, gw_ref, uw_ref, dw_ref, y_ref):
    b = pl.program_id(0)

    @pl.when(ba_ref[b] == 1)
    def _():
        x = x_ref[...].astype(jnp.bfloat16)
        g = jnp.dot(x, gw_ref[0].astype(jnp.bfloat16),
                    preferred_element_type=jnp.float32)
        u = jnp.dot(x, uw_ref[0].astype(jnp.bfloat16),
                    preferred_element_type=jnp.float32)
        h = ((g * lax.logistic(g)) * u).astype(jnp.bfloat16)
        y_ref[...] = jnp.dot(h, dw_ref[0].astype(jnp.bfloat16),
                             preferred_element_type=jnp.float32)


def _grouped_gemm(x_s, gate_w, up_w, down_w, block_expert, block_active):
    def _rowmap(b, be, ba):
        return (jnp.where(ba[b] == 1, b, NB - 1), 0)

    grid_spec = pltpu.PrefetchScalarGridSpec(
        num_scalar_prefetch=2,
        grid=(NB,),
        in_specs=[
            pl.BlockSpec((BT, EMBED), _rowmap),
            pl.BlockSpec((1, EMBED, FF), lambda b, be, ba: (be[b], 0, 0)),
            pl.BlockSpec((1, EMBED, FF), lambda b, be, ba: (be[b], 0, 0)),
            pl.BlockSpec((1, FF, EMBED), lambda b, be, ba: (be[b], 0, 0)),
        ],
        out_specs=pl.BlockSpec((BT, EMBED), _rowmap),
    )
    return pl.pallas_call(
        _gemm_body,
        grid_spec=grid_spec,
        out_shape=jax.ShapeDtypeStruct((NPAD, EMBED), jnp.float32),
        compiler_params=pltpu.CompilerParams(
            dimension_semantics=("arbitrary",),
        ),
    )(block_expert, block_active, x_s, gate_w, up_w, down_w)


def kernel(x, router_w, gate_w, up_w, down_w):
    B, S, D = x.shape
    x_flat = x.reshape(T, D)
    e1, e2, p1, p2 = _router(x_flat, router_w)
    pos_by_pair, block_expert, block_active = _plan(e1, e2)
    tok_pair = jnp.tile(jnp.arange(T, dtype=jnp.int32), 2)
    x_s = _sc_gather(x_flat, tok_pair, pos_by_pair)
    y = _grouped_gemm(x_s, gate_w, up_w, down_w, block_expert, block_active)
    g0, g1 = _sc_combine(y, pos_by_pair[:T], pos_by_pair[T:])
    out = p1[:, None] * g0 + p2[:, None] * g1
    return out.reshape(B, S, D)


# speedup vs baseline: 2.0478x; 1.0069x over previous
import functools

import jax
import jax.numpy as jnp
from jax import lax
from jax.experimental import pallas as pl
from jax.experimental.pallas import tpu as pltpu
from jax.experimental.pallas import tpu_sc as plsc

EMBED = 1024
FF = 2048
NE = 8
TOPK = 2
T = 2048
NPAIR = T * TOPK
BT = 256
NB = NPAIR // BT + NE
NPAD = NB * BT
LANES = 128


def _router_body(x_ref, rw_ref, e1_ref, e2_ref, p1_ref, p2_ref):
    logits = jnp.dot(x_ref[...], rw_ref[...], preferred_element_type=jnp.float32)
    lane = lax.broadcasted_iota(jnp.int32, (T, LANES), 1)
    neg = jnp.float32(-1e30)
    logits = jnp.where(lane < NE, logits, neg)
    m1 = jnp.max(logits, axis=1, keepdims=True)
    i1 = jnp.min(jnp.where(logits == m1, lane, LANES), axis=1, keepdims=True)
    l2 = jnp.where(lane == i1, neg, logits)
    m2 = jnp.max(l2, axis=1, keepdims=True)
    i2 = jnp.min(jnp.where(l2 == m2, lane, LANES), axis=1, keepdims=True)
    p1 = 1.0 / (1.0 + jnp.exp(m2 - m1))
    p2 = 1.0 - p1
    zero = jnp.zeros((T, LANES), jnp.int32)
    e1_ref[...] = zero + i1
    e2_ref[...] = zero + i2
    p1_ref[...] = jnp.zeros((T, LANES), jnp.float32) + p1
    p2_ref[...] = jnp.zeros((T, LANES), jnp.float32) + p2


def _router(x_flat, router_w):
    rw_pad = jnp.zeros((EMBED, LANES), jnp.float32).at[:, :NE].set(router_w)
    outs = pl.pallas_call(
        _router_body,
        out_shape=(
            jax.ShapeDtypeStruct((T, LANES), jnp.int32),
            jax.ShapeDtypeStruct((T, LANES), jnp.int32),
            jax.ShapeDtypeStruct((T, LANES), jnp.float32),
            jax.ShapeDtypeStruct((T, LANES), jnp.float32),
        ),
    )(x_flat, rw_pad)
    e1, e2, p1, p2 = (o[:, 0] for o in outs)
    return e1, e2, p1, p2


def _plan(e1, e2):
    keys = jnp.concatenate([e1, e2])
    onehot = (keys[:, None] == jnp.arange(NE, dtype=jnp.int32)[None, :]).astype(jnp.int32)
    csum = jnp.cumsum(onehot, axis=0)
    rank = jnp.sum(onehot * csum, axis=1) - 1
    counts = csum[-1]
    nblk = (counts + BT - 1) // BT
    blk_start = jnp.cumsum(nblk) - nblk
    num_used = jnp.sum(nblk)
    pos_by_pair = (jnp.sum(onehot * blk_start[None, :], axis=1) * BT + rank).astype(jnp.int32)
    blk_end = jnp.cumsum(nblk)
    bidx = jnp.arange(NB, dtype=jnp.int32)
    block_expert = jnp.clip(
        jnp.sum((blk_end[None, :] <= bidx[:, None]).astype(jnp.int32), axis=1),
        0, NE - 1).astype(jnp.int32)
    block_active = (bidx < num_used).astype(jnp.int32)
    return pos_by_pair, block_expert, block_active


NW = 32
DCH = 32
NDC = NPAIR // NW // DCH
CCH = 16
NCC = T // NW // CCH


@functools.lru_cache(maxsize=None)
def _sc_dispatch_k():
    mesh = plsc.VectorSubcoreMesh(core_axis_name="c", subcore_axis_name="s")

    @functools.partial(
        pl.kernel,
        mesh=mesh,
        out_type=jax.ShapeDtypeStruct((NPAD, EMBED), jnp.float32),
        scratch_types=[
            pltpu.VMEM((2, DCH), jnp.int32),
            pltpu.VMEM((2, DCH), jnp.int32),
            pltpu.VMEM((2, DCH, EMBED), jnp.float32),
            pltpu.SemaphoreType.DMA,
            pltpu.SemaphoreType.DMA,
            pltpu.SemaphoreType.DMA,
            pltpu.SemaphoreType.DMA,
        ],
    )
    def k(x_hbm, tok_hbm, pos_hbm, xs_hbm, tok_v, pos_v, rows_v,
          g0, g1, s0, s1):
        wid = lax.axis_index("s") * 2 + lax.axis_index("c")
        per_w = NPAIR // NW
        gsem = (g0, g1)
        ssem = (s0, s1)

        def start_gather(c):
            s = c % 2
            base = wid * per_w + c * DCH
            pltpu.sync_copy(tok_hbm.at[pl.ds(base, DCH)], tok_v.at[s])
            pltpu.sync_copy(pos_hbm.at[pl.ds(base, DCH)], pos_v.at[s])
            return pltpu.async_copy(x_hbm.at[tok_v.at[s]], rows_v.at[s], gsem[s])

        gh = [start_gather(0), start_gather(1)]
        sh = [None, None]
        for c in range(NDC):
            s = c % 2
            gh[s].wait()
            sh[s] = pltpu.async_copy(rows_v.at[s], xs_hbm.at[pos_v.at[s]], ssem[s])
            if c + 2 < NDC:
                sh[s].wait()
                gh[s] = start_gather(c + 2)
        for c in range(max(0, NDC - 2), NDC):
            sh[c % 2].wait()

    return k


@functools.lru_cache(maxsize=None)
def _sc_combine_k():
    mesh = plsc.VectorSubcoreMesh(core_axis_name="c", subcore_axis_name="s")

    @functools.partial(
        pl.kernel,
        mesh=mesh,
        out_type=(
            jax.ShapeDtypeStruct((T, EMBED), jnp.float32),
            jax.ShapeDtypeStruct((T, EMBED), jnp.float32),
        ),
        scratch_types=[
            pltpu.VMEM((2, CCH), jnp.int32),
            pltpu.VMEM((2, CCH), jnp.int32),
            pltpu.VMEM((2, CCH, EMBED), jnp.float32),
            pltpu.VMEM((2, CCH, EMBED), jnp.float32),
            pltpu.SemaphoreType.DMA,
            pltpu.SemaphoreType.DMA,
            pltpu.SemaphoreType.DMA,
            pltpu.SemaphoreType.DMA,
            pltpu.SemaphoreType.DMA,
            pltpu.SemaphoreType.DMA,
            pltpu.SemaphoreType.DMA,
            pltpu.SemaphoreType.DMA,
        ],
    )
    def k(y_hbm, i0_hbm, i1_hbm, o0_hbm, o1_hbm, i0_v, i1_v, a_v, b_v,
          ga0, ga1, gb0, gb1, wa0, wa1, wb0, wb1):
        wid = lax.axis_index("s") * 2 + lax.axis_index("c")
        per_w = T // NW
        gasem = (ga0, ga1)
        gbsem = (gb0, gb1)
        wasem = (wa0, wa1)
        wbsem = (wb0, wb1)

        def start_gathers(c):
            s = c % 2
            base = wid * per_w + c * CCH
            pltpu.sync_copy(i0_hbm.at[pl.ds(base, CCH)], i0_v.at[s])
            pltpu.sync_copy(i1_hbm.at[pl.ds(base, CCH)], i1_v.at[s])
            return (pltpu.async_copy(y_hbm.at[i0_v.at[s]], a_v.at[s], gasem[s]),
                    pltpu.async_copy(y_hbm.at[i1_v.at[s]], b_v.at[s], gbsem[s]))

        gh = [start_gathers(0), start_gathers(1)]
        wh = [None, None]
        for c in range(NCC):
            s = c % 2
            base = wid * per_w + c * CCH
            gh[s][0].wait()
            gh[s][1].wait()
            wh[s] = (pltpu.async_copy(a_v.at[s], o0_hbm.at[pl.ds(base, CCH)], wasem[s]),
                     pltpu.async_copy(b_v.at[s], o1_hbm.at[pl.ds(base, CCH)], wbsem[s]))
            if c + 2 < NCC:
                wh[s][0].wait()
                wh[s][1].wait()
                gh[s] = start_gathers(c + 2)
        for c in range(max(0, NCC - 2), NCC):
            wh[c % 2][0].wait()
            wh[c % 2][1].wait()

    return k


def _sc_gather(x_flat, tok_pair, pos_by_pair):
    return _sc_dispatch_k()(x_flat, tok_pair, pos_by_pair)


def _sc_combine(y, i0, i1):
    return _sc_combine_k()(y, i0, i1)


def _gemm_body(be_ref, ba_ref, x_ref, gw_ref, uw_ref, dw_ref, y_ref):
    b = pl.program_id(0)

    @pl.when(ba_ref[b] == 1)
    def _():
        x = x_ref[...]
        g = jnp.dot(x, gw_ref[0], preferred_element_type=jnp.float32)
        u = jnp.dot(x, uw_ref[0], preferred_element_type=jnp.float32)
        h = (g * lax.logistic(g)) * u
        y_ref[...] = jnp.dot(h, dw_ref[0], preferred_element_type=jnp.float32)


def _grouped_gemm(x_s, gate_w, up_w, down_w, block_expert, block_active):
    def _rowmap(b, be, ba):
        return (jnp.where(ba[b] == 1, b, NB - 1), 0)

    grid_spec = pltpu.PrefetchScalarGridSpec(
        num_scalar_prefetch=2,
        grid=(NB,),
        in_specs=[
            pl.BlockSpec((BT, EMBED), _rowmap),
            pl.BlockSpec((1, EMBED, FF), lambda b, be, ba: (be[b], 0, 0)),
            pl.BlockSpec((1, EMBED, FF), lambda b, be, ba: (be[b], 0, 0)),
            pl.BlockSpec((1, FF, EMBED), lambda b, be, ba: (be[b], 0, 0)),
        ],
        out_specs=pl.BlockSpec((BT, EMBED), _rowmap),
    )
    return pl.pallas_call(
        _gemm_body,
        grid_spec=grid_spec,
        out_shape=jax.ShapeDtypeStruct((NPAD, EMBED), jnp.float32),
        compiler_params=pltpu.CompilerParams(
            dimension_semantics=("arbitrary",),
        ),
    )(block_expert, block_active, x_s, gate_w, up_w, down_w)


def kernel(x, router_w, gate_w, up_w, down_w):
    B, S, D = x.shape
    x_flat = x.reshape(T, D)
    e1, e2, p1, p2 = _router(x_flat, router_w)
    pos_by_pair, block_expert, block_active = _plan(e1, e2)
    tok_pair = jnp.tile(jnp.arange(T, dtype=jnp.int32), 2)
    x_s = _sc_gather(x_flat, tok_pair, pos_by_pair)
    y = _grouped_gemm(x_s, gate_w, up_w, down_w, block_expert, block_active)
    g0, g1 = _sc_combine(y, pos_by_pair[:T], pos_by_pair[T:])
    out = p1[:, None] * g0 + p2[:, None] * g1
    return out.reshape(B, S, D)
